# baseline (speedup 1.0000x reference)
import functools

import jax
import jax.numpy as jnp
from jax import lax
from jax.experimental import pallas as pl
from jax.experimental.pallas import tpu as pltpu
from jax.experimental.pallas import tpu_sc as plsc

_VOCAB = 100000
_D = 64
_B = 4
_S = 2048

_info = plsc.get_sparse_core_info()
_NC = _info.num_cores
_NS = _info.num_subcores
_L = _info.num_lanes
_NW = _NC * _NS
_DPW = _D // _NW

_mesh = plsc.VectorSubcoreMesh(core_axis_name="c", subcore_axis_name="s")


@functools.partial(
    pl.kernel,
    out_type=jax.ShapeDtypeStruct((_B, _D, _S), jnp.float32),
    mesh=_mesh,
    compiler_params=pltpu.CompilerParams(needs_layout_passes=False),
    scratch_types=[
        pltpu.VMEM((_VOCAB,), jnp.float32),
        pltpu.VMEM((_B, _S), jnp.int32),
        pltpu.VMEM((_DPW, _S), jnp.float32),
        pltpu.VMEM((_DPW, _B, _S), jnp.float32),
        pltpu.SemaphoreType.DMA,
        pltpu.SemaphoreType.DMA,
        pltpu.SemaphoreType.DMA,
        pltpu.SemaphoreType.REGULAR,
    ],
)
def _embed(x_hbm, tokT_hbm, posT_hbm, out_hbm, row_v, idx_v, pos_v, out_v,
           sem_row, sem_in, sem_out, sem_fence):
    d0 = (lax.axis_index("c") * _NS + lax.axis_index("s")) * _DPW

    cp_row0 = pltpu.async_copy(tokT_hbm.at[d0], row_v, sem_row)
    cp_idx = pltpu.async_copy(x_hbm, idx_v, sem_in)
    cp_pos = pltpu.async_copy(posT_hbm.at[pl.ds(d0, _DPW)], pos_v, sem_in)

    def compute(k, par):
        def body(i):
            base = i * _L
            pv = pos_v[k, pl.ds(base, _L)]
            for b in range(_B):
                ids = idx_v[b, pl.ds(base, _L)]
                g = plsc.load_gather(row_v, [ids])
                out_v[k, b, pl.ds(base, _L)] = g + pv

        if par:
            plsc.parallel_loop(0, _S // _L, unroll=8)(body)
        else:
            lax.fori_loop(0, _S // _L, lambda i, _: (body(i), ())[1], (),
                          unroll=8)

    cp_idx.wait()
    cp_pos.wait()
    cp_row0.wait()
    compute(0, par=True)
    pl.semaphore_signal(sem_fence, 1)
    pl.semaphore_wait(sem_fence, 1)
    cp_row1 = pltpu.async_copy(tokT_hbm.at[d0 + 1], row_v, sem_row)
    outs0 = [
        pltpu.async_copy(out_v.at[0, b], out_hbm.at[b, d0], sem_out)
        for b in range(_B)
    ]
    cp_row1.wait()
    compute(1, par=True)
    for b in range(_B):
        pltpu.sync_copy(out_v.at[1, b], out_hbm.at[b, d0 + 1])
    for cp in outs0:
        cp.wait()


def kernel(x, token_table, pos_table):
    out = _embed(x.astype(jnp.int32), token_table.T, pos_table.T)
    return out.transpose(0, 2, 1)


# speedup vs baseline: 1.0277x; 1.0277x over previous
import functools

import jax
import jax.numpy as jnp
from jax import lax
from jax.experimental import pallas as pl
from jax.experimental.pallas import tpu as pltpu
from jax.experimental.pallas import tpu_sc as plsc

_VOCAB = 100000
_D = 64
_B = 4
_S = 2048

_info = plsc.get_sparse_core_info()
_NC = _info.num_cores
_NS = _info.num_subcores
_L = _info.num_lanes
_NW = _NC * _NS
_DPW = _D // _NW

_mesh = plsc.VectorSubcoreMesh(core_axis_name="c", subcore_axis_name="s")


@functools.partial(
    pl.kernel,
    out_type=jax.ShapeDtypeStruct((_B, _D, _S), jnp.float32),
    mesh=_mesh,
    compiler_params=pltpu.CompilerParams(needs_layout_passes=False),
    scratch_types=[
        pltpu.VMEM((_VOCAB,), jnp.float32),
        pltpu.VMEM((_B, _S), jnp.int32),
        pltpu.VMEM((_DPW, _S), jnp.float32),
        pltpu.VMEM((_DPW, _B, _S), jnp.float32),
        pltpu.SemaphoreType.DMA,
        pltpu.SemaphoreType.DMA,
        pltpu.SemaphoreType.DMA,
        pltpu.SemaphoreType.REGULAR,
    ],
)
def _embed(x_hbm, tokT_hbm, posT_hbm, out_hbm, row_v, idx_v, pos_v, out_v,
           sem_row, sem_in, sem_out, sem_fence):
    d0 = (lax.axis_index("c") * _NS + lax.axis_index("s")) * _DPW

    cp_row0 = pltpu.async_copy(tokT_hbm.at[d0], row_v, sem_row)
    cp_idx = pltpu.async_copy(x_hbm, idx_v, sem_in)
    cp_pos = pltpu.async_copy(posT_hbm.at[pl.ds(d0, _DPW)], pos_v, sem_in)

    def compute(k):
        @functools.partial(plsc.parallel_loop, 0, _S // _L, unroll=8)
        def _body(i):
            base = i * _L
            pv = pos_v[k, pl.ds(base, _L)]
            for b in range(_B):
                ids = idx_v[b, pl.ds(base, _L)]
                g = plsc.load_gather(row_v, [ids])
                out_v[k, b, pl.ds(base, _L)] = g + pv
        pl.semaphore_signal(sem_fence, 1)
        pl.semaphore_wait(sem_fence, 1)

    cp_idx.wait()
    cp_pos.wait()
    cp_row0.wait()
    compute(0)
    cp_row1 = pltpu.async_copy(tokT_hbm.at[d0 + 1], row_v, sem_row)
    outs0 = [
        pltpu.async_copy(out_v.at[0, b], out_hbm.at[b, d0], sem_out)
        for b in range(_B)
    ]
    cp_row1.wait()
    compute(1)
    for b in range(_B):
        pltpu.sync_copy(out_v.at[1, b], out_hbm.at[b, d0 + 1])
    for cp in outs0:
        cp.wait()


def kernel(x, token_table, pos_table):
    out = _embed(x.astype(jnp.int32), token_table.T, pos_table.T)
    return out.transpose(0, 2, 1)
